# Initial kernel scaffold; baseline (speedup 1.0000x reference)
import jax
import jax.numpy as jnp
from jax.experimental import pallas as pl


def kernel(x, idx):
    raise NotImplementedError("write your pallas kernel here")


# speedup vs baseline: 9.6283x; 9.6283x over previous
import jax
import jax.numpy as jnp
from jax import lax
from jax.experimental import pallas as pl
from jax.experimental.pallas import tpu as pltpu
from jax.experimental.pallas import tpu_sc as plsc

_B = 64
_T = 262144
_NS = 2048
_NC = 2
_NSUB = 16
_NW = _NC * _NSUB
_ROWS_PER_W = _B // _NW
_CH = 32768
_NCHUNK = _T // _CH
_NBUF = 3


def _scatter_chunk(buf, idx_v, lo):
    zeros16 = jnp.zeros((16,), jnp.float32)
    lo_v = jnp.full((16,), lo, jnp.int32)

    def body(k, carry):
        iv = idx_v[pl.ds(k * 16, 16)]
        t = iv - lo_v
        m = plsc.bitcast(t, jnp.uint32) < jnp.full((16,), _CH, jnp.uint32)
        plsc.store_scatter(buf, [t], zeros16, mask=m)
        return carry

    lax.fori_loop(0, _NS // 16, body, 0, unroll=8)


def _sc_body(x_hbm, idx_hbm, out_hbm, bufs, idx_v, rsem, wsem, isem):
    wid = lax.axis_index("s") * _NC + lax.axis_index("c")
    row0 = wid * _ROWS_PER_W

    for i in range(_ROWS_PER_W):
        b = row0 + i
        pltpu.async_copy(idx_hbm.at[pl.ds(b * _NS, _NS)], idx_v, isem).wait()
        rd = {0: pltpu.async_copy(
            x_hbm.at[pl.ds(b * _T, _CH)], bufs[0], rsem)}
        wr = {}
        for c in range(_NCHUNK):
            if c >= 2:
                wr[c - 2].wait()
            if c + 1 < _NCHUNK:
                rd[c + 1] = pltpu.async_copy(
                    x_hbm.at[pl.ds(b * _T + (c + 1) * _CH, _CH)],
                    bufs[(c + 1) % _NBUF], rsem)
            rd[c].wait()
            _scatter_chunk(bufs[c % _NBUF], idx_v, c * _CH)
            wr[c] = pltpu.async_copy(
                bufs[c % _NBUF], out_hbm.at[pl.ds(b * _T + c * _CH, _CH)], wsem)
        wr[_NCHUNK - 2].wait()
        wr[_NCHUNK - 1].wait()


@jax.jit
def _sc_cut(x2, idx2):
    mesh = plsc.VectorSubcoreMesh(
        core_axis_name="c", subcore_axis_name="s",
        num_cores=_NC, num_subcores=_NSUB,
    )

    def body(x_hbm, idx_hbm, out_hbm, b0, b1, b2, idx_v, rsem, wsem, isem):
        _sc_body(x_hbm, idx_hbm, out_hbm, (b0, b1, b2), idx_v,
                 rsem, wsem, isem)

    return pl.kernel(
        body,
        out_type=jax.ShapeDtypeStruct((_B * _T,), jnp.float32),
        mesh=mesh,
        compiler_params=pltpu.CompilerParams(needs_layout_passes=False),
        scratch_types=[
            pltpu.VMEM((_CH,), jnp.float32),
            pltpu.VMEM((_CH,), jnp.float32),
            pltpu.VMEM((_CH,), jnp.float32),
            pltpu.VMEM((_NS,), jnp.int32),
            pltpu.SemaphoreType.DMA,
            pltpu.SemaphoreType.DMA,
            pltpu.SemaphoreType.DMA,
        ],
    )(x2, idx2)


def kernel(x, idx):
    Bb, Tt, Cc = x.shape
    out = _sc_cut(x.reshape(Bb * Tt), idx.reshape(Bb * _NS))
    return out.reshape(Bb, Tt, Cc)
